# baseline (speedup 1.0000x reference)
import jax
import jax.numpy as jnp
import numpy as np
from jax import lax
from jax.experimental import pallas as pl
from jax.experimental.pallas import tpu as pltpu
from jax.experimental.pallas import tpu_sc as plsc

_N = 10000
_E = 320000
_SH = 180
_NRBF = 18
_W = 256
_NC, _NS = 2, 16
_NW = _NC * _NS
_NSPLIT = 5
_ES = _E // _NSPLIT
_EPW = _ES // _NW
_CHG = 80
_NCHG = _EPW // _CHG
_CH = 40
_NCH = _EPW // _CH
_NBUF = 5
_RPT = 624
_REM = _N - _RPT * _NS
_BE = 2000
_BN = 2000


def _mesh():
    return plsc.VectorSubcoreMesh(core_axis_name="c", subcore_axis_name="s",
                                  num_cores=_NC, num_subcores=_NS)


def _sc_gather(table, sidx3, didx3):
    def body(table_hbm, sidx_hbm, didx_hbm, gs_hbm, gd_hbm, idx_v, rows_v, sem):
        c = lax.axis_index("c")
        s = lax.axis_index("s")
        wid = s * _NC + c
        for idx_hbm, out_hbm in ((sidx_hbm, gs_hbm), (didx_hbm, gd_hbm)):
            pltpu.sync_copy(idx_hbm.at[wid], idx_v)

            def block(t, _):
                j0 = t * _NBUF
                descs = [
                    pltpu.async_copy(table_hbm.at[idx_v.at[j0 + b]],
                                     rows_v.at[b], sem)
                    for b in range(_NBUF)
                ]
                for b in range(_NBUF):
                    descs[b].wait()
                    pltpu.sync_copy(
                        rows_v.at[b],
                        out_hbm.at[pl.ds(wid * _EPW + (j0 + b) * _CHG, _CHG)])
                return 0

            lax.fori_loop(0, _NCHG // _NBUF, block, 0)

    return pl.kernel(
        body,
        out_type=(jax.ShapeDtypeStruct((_ES, _W), jnp.float32),
                  jax.ShapeDtypeStruct((_ES, _W), jnp.float32)),
        mesh=_mesh(),
        scratch_types=[pltpu.VMEM((_NCHG, _CHG), jnp.int32),
                       pltpu.VMEM((_NBUF, _CHG, _W), jnp.float32),
                       pltpu.SemaphoreType.DMA],
    )(table, sidx3, didx3)


def _sc_scatter(dout, didx3, zeros):
    def body(dout_hbm, didx_hbm, zeros_hbm, acc_hbm, idx_v, rows_v, acc_sh,
             sem, sem2):
        c = lax.axis_index("c")
        s = lax.axis_index("s")
        wid = s * _NC + c
        pltpu.sync_copy(didx_hbm.at[wid], idx_v)
        for k in range(_W // 128):
            pltpu.sync_copy(zeros_hbm, acc_sh.at[pl.ds(s * _RPT, _RPT)])

            @pl.when(s == 0)
            def _():
                pltpu.sync_copy(zeros_hbm.at[pl.ds(0, _REM)],
                                acc_sh.at[pl.ds(_RPT * _NS, _REM)])

            plsc.subcore_barrier()

            def block(t, _):
                j0 = t * _NBUF
                descs = [
                    pltpu.async_copy(
                        dout_hbm.at[pl.ds(wid * _EPW + (j0 + b) * _CH, _CH),
                                    pl.ds(k * 128, 128)], rows_v.at[b], sem)
                    for b in range(_NBUF)
                ]
                adds = []
                for b in range(_NBUF):
                    descs[b].wait()
                    adds.append(pltpu.async_copy(
                        rows_v.at[b], acc_sh.at[idx_v.at[j0 + b]], sem2,
                        add=True))
                for a in adds:
                    a.wait()
                return 0

            lax.fori_loop(0, _NCH // _NBUF, block, 0)
            plsc.subcore_barrier()
            pltpu.sync_copy(acc_sh.at[pl.ds(s * _RPT, _RPT)],
                            acc_hbm.at[c, pl.ds(s * _RPT, _RPT),
                                       pl.ds(k * 128, 128)])

            @pl.when(s == 0)
            def _():
                pltpu.sync_copy(acc_sh.at[pl.ds(_RPT * _NS, _REM)],
                                acc_hbm.at[c, pl.ds(_RPT * _NS, _REM),
                                           pl.ds(k * 128, 128)])

            plsc.subcore_barrier()

    return pl.kernel(
        body,
        out_type=jax.ShapeDtypeStruct((_NC, _N, _W), jnp.float32),
        mesh=_mesh(),
        scratch_types=[pltpu.VMEM((_NCH, _CH), jnp.int32),
                       pltpu.VMEM((_NBUF, _CH, 128), jnp.float32),
                       pltpu.VMEM_SHARED((_N, 128), jnp.float32),
                       pltpu.SemaphoreType.DMA,
                       pltpu.SemaphoreType.DMA],
    )(dout, didx3, zeros)


def _silu(v):
    return v * jax.nn.sigmoid(v)


def _encoder(x, pos, ln_g, ln_b, w0, b0, w1, b1):
    def body(x_ref, pos_ref, g_ref, b_ref, w0_ref, b0_ref, w1_ref, b1_ref,
             out_ref):
        xv = x_ref[...]
        mu = jnp.mean(xv, axis=1, keepdims=True)
        var = jnp.mean((xv - mu) ** 2, axis=1, keepdims=True)
        h = (xv - mu) / jnp.sqrt(var + 1e-5) * g_ref[...] + b_ref[...]
        h = _silu(h @ w0_ref[...] + b0_ref[...])
        h = _silu(h @ w1_ref[...] + b1_ref[...])
        out_ref[...] = jnp.concatenate(
            [h, pos_ref[...], jnp.zeros((_BN, _W - _SH - 3), jnp.float32)],
            axis=1)

    full = lambda shape: pl.BlockSpec(shape, lambda i: (0,) * len(shape))
    return pl.pallas_call(
        body,
        grid=(_N // _BN,),
        in_specs=[pl.BlockSpec((_BN, 128), lambda i: (i, 0)),
                  pl.BlockSpec((_BN, 3), lambda i: (i, 0)),
                  full((1, 128)), full((1, 128)),
                  full((128, _SH // 2)), full((1, _SH // 2)),
                  full((_SH // 2, _SH)), full((1, _SH))],
        out_specs=pl.BlockSpec((_BN, _W), lambda i: (i, 0)),
        out_shape=jax.ShapeDtypeStruct((_N, _W), jnp.float32),
    )(x, pos, ln_g, ln_b, w0, b0, w1, b1)


_WIDTH = np.float32(30.0 / _NRBF)


def _edge_mlp(gs, gd, ea, w1a, w1b, w1c, w1d, b1, w2, b2, w3, b3,
              gw1, gb1, gw2, gb2, hw, hb, xw1, xb1, xw2, xb2):
    def body(gs_ref, gd_ref, ea_ref, w1a_ref, w1b_ref, w1c_ref, w1d_ref,
             b1_ref, w2_ref, b2_ref, w3_ref, b3_ref, gw1_ref, gb1_ref,
             gw2_ref, gb2_ref, hw_ref, hb_ref, xw1_ref, xb1_ref, xw2_ref,
             xb2_ref, out_ref):
        gs_v = gs_ref[...]
        gd_v = gd_ref[...]
        hs = gs_v[:, :_SH]
        hd = gd_v[:, :_SH]
        r = gs_v[:, _SH:_SH + 3] - gd_v[:, _SH:_SH + 3]
        d = jnp.sqrt(jnp.sum(r * r, axis=1, keepdims=True) + 1e-12)
        centers = lax.broadcasted_iota(jnp.int32, (1, _NRBF), 1).astype(
            jnp.float32) * np.float32(30.0 / (_NRBF - 1))
        z = (d - centers) / (_WIDTH + np.float32(1e-8))
        rbf = jnp.exp(-(z * z))
        m = (hs @ w1a_ref[...] + hd @ w1b_ref[...] + rbf @ w1c_ref[...]
             + ea_ref[...] @ w1d_ref[...] + b1_ref[...])
        m = _silu(m)
        m = _silu(m @ w2_ref[...] + b2_ref[...])
        m = _silu(m @ w3_ref[...] + b3_ref[...])
        g1 = jax.nn.relu(m @ gw1_ref[...] + gb1_ref[...])
        a = jax.nn.sigmoid(g1 @ gw2_ref[...] + gb2_ref[...])
        m = m * a
        dh = _silu(m @ hw_ref[...] + hb_ref[...])
        coeff = (_silu(m @ xw1_ref[...] + xb1_ref[...]) @ xw2_ref[...]
                 + xb2_ref[...]) * np.float32(0.08)
        out_ref[...] = jnp.concatenate(
            [dh, r * coeff, jnp.zeros((_BE, _W - _SH - 3), jnp.float32)],
            axis=1)

    full = lambda shape: pl.BlockSpec(shape, lambda i: (0,) * len(shape))
    return pl.pallas_call(
        body,
        grid=(_ES // _BE,),
        in_specs=[pl.BlockSpec((_BE, _W), lambda i: (i, 0)),
                  pl.BlockSpec((_BE, _W), lambda i: (i, 0)),
                  pl.BlockSpec((_BE, 16), lambda i: (i, 0)),
                  full((_SH, 320)), full((_SH, 320)), full((_NRBF, 320)),
                  full((16, 320)), full((1, 320)),
                  full((320, 160)), full((1, 160)),
                  full((160, 128)), full((1, 128)),
                  full((128, 64)), full((1, 64)),
                  full((64, 1)), full((1, 1)),
                  full((128, _SH)), full((1, _SH)),
                  full((128, 32)), full((1, 32)),
                  full((32, 1)), full((1, 1))],
        out_specs=pl.BlockSpec((_BE, _W), lambda i: (i, 0)),
        out_shape=jax.ShapeDtypeStruct((_ES, _W), jnp.float32),
    )(gs, gd, ea, w1a, w1b, w1c, w1d, b1, w2, b2, w3, b3,
      gw1, gb1, gw2, gb2, hw, hb, xw1, xb1, xw2, xb2)


def _node_update(table, accs, lng, lnb, alpha_i):
    na = len(accs)

    def body(*refs):
        t_ref = refs[0]
        acc_refs = refs[1:1 + na]
        g_ref, b_ref, al_ref, out_ref = refs[1 + na:]
        t = t_ref[...]
        asum = acc_refs[0][...]
        for ar in acc_refs[1:]:
            asum = asum + ar[...]
        sg = jax.nn.sigmoid(al_ref[0, 0])
        u = t[:, :_SH] + sg * asum[:, :_SH]
        mu = jnp.mean(u, axis=1, keepdims=True)
        var = jnp.mean((u - mu) ** 2, axis=1, keepdims=True)
        hn = (u - mu) / jnp.sqrt(var + 1e-5) * g_ref[...] + b_ref[...]
        xcn = t[:, _SH:_SH + 3] + asum[:, _SH:_SH + 3]
        out_ref[...] = jnp.concatenate(
            [hn, xcn, jnp.zeros((_BN, _W - _SH - 3), jnp.float32)], axis=1)

    full = lambda shape: pl.BlockSpec(shape, lambda i: (0,) * len(shape))
    return pl.pallas_call(
        body,
        grid=(_N // _BN,),
        in_specs=[pl.BlockSpec((_BN, _W), lambda i: (i, 0))] * (1 + na)
                 + [full((1, _SH)), full((1, _SH)), full((1, 1))],
        out_specs=pl.BlockSpec((_BN, _W), lambda i: (i, 0)),
        out_shape=jax.ShapeDtypeStruct((_N, _W), jnp.float32),
    )(table, *accs, lng, lnb, alpha_i)


def kernel(x, pos, edge_attr, edge_index, ln_g, ln_b, W0, b0, W1, b1,
           Ew1, Eb1, Ew2, Eb2, Ew3, Eb3, Gw1, Gb1, Gw2, Gb2,
           Hw, Hb, Xw1, Xb1, Xw2, Xb2, LNg, LNb, alpha):
    sidxg = edge_index[0].reshape(_NSPLIT, _NW, _NCHG, _CHG)
    didxg = edge_index[1].reshape(_NSPLIT, _NW, _NCHG, _CHG)
    didx = edge_index[1].reshape(_NSPLIT, _NW, _NCH, _CH)
    ea = edge_attr.reshape(_NSPLIT, _ES, edge_attr.shape[1])
    zeros = jnp.zeros((_RPT, 128), jnp.float32)
    table = _encoder(x, pos, ln_g.reshape(1, -1), ln_b.reshape(1, -1),
                     W0, b0.reshape(1, -1), W1, b1.reshape(1, -1))
    for i in range(Ew1.shape[0]):
        w1a = Ew1[i, :_SH]
        w1b = Ew1[i, _SH:2 * _SH]
        w1c = Ew1[i, 2 * _SH:2 * _SH + _NRBF]
        w1d = Ew1[i, 2 * _SH + _NRBF:]
        accs = []
        for sp in range(_NSPLIT):
            gs, gd = _sc_gather(table, sidxg[sp], didxg[sp])
            dout = _edge_mlp(gs, gd, ea[sp], w1a, w1b, w1c, w1d,
                             Eb1[i].reshape(1, -1), Ew2[i],
                             Eb2[i].reshape(1, -1),
                             Ew3[i], Eb3[i].reshape(1, -1),
                             Gw1[i], Gb1[i].reshape(1, -1),
                             Gw2[i], Gb2[i].reshape(1, -1),
                             Hw[i], Hb[i].reshape(1, -1),
                             Xw1[i], Xb1[i].reshape(1, -1),
                             Xw2[i], Xb2[i].reshape(1, -1))
            acc = _sc_scatter(dout, didx[sp], zeros)
            accs.extend([acc[0], acc[1]])
        table = _node_update(table, accs, LNg[i].reshape(1, -1),
                             LNb[i].reshape(1, -1), alpha[i].reshape(1, 1))
    return table[:, :_SH], table[:, _SH:_SH + 3]


# speedup vs baseline: 1.0621x; 1.0621x over previous
import jax
import jax.numpy as jnp
import numpy as np
from jax import lax
from jax.experimental import pallas as pl
from jax.experimental.pallas import tpu as pltpu
from jax.experimental.pallas import tpu_sc as plsc

_N = 10000
_E = 320000
_SH = 180
_NRBF = 18
_W = 256
_NC, _NS = 2, 16
_NW = _NC * _NS
_NSPLIT = 5
_ES = _E // _NSPLIT
_EPW = _ES // _NW
_CHG = 40
_NCHG = _EPW // _CHG
_CH = 40
_NCH = _EPW // _CH
_NBUF = 5
_RPT = 624
_REM = _N - _RPT * _NS
_BE = 4000
_BN = 2000


def _mesh():
    return plsc.VectorSubcoreMesh(core_axis_name="c", subcore_axis_name="s",
                                  num_cores=_NC, num_subcores=_NS)


def _sc_gather(table, sidx3, didx3):
    def body(table_hbm, sidx_hbm, didx_hbm, gs_hbm, gd_hbm, idx_v, rows_v, sem):
        c = lax.axis_index("c")
        s = lax.axis_index("s")
        wid = s * _NC + c
        for idx_hbm, out_hbm in ((sidx_hbm, gs_hbm), (didx_hbm, gd_hbm)):
            pltpu.sync_copy(idx_hbm.at[wid], idx_v)

            def block(t, _):
                j0 = t * _NBUF
                descs = [
                    pltpu.async_copy(table_hbm.at[idx_v.at[j0 + b]],
                                     rows_v.at[b], sem)
                    for b in range(_NBUF)
                ]
                for b in range(_NBUF):
                    descs[b].wait()
                    pltpu.sync_copy(
                        rows_v.at[b],
                        out_hbm.at[pl.ds(wid * _EPW + (j0 + b) * _CHG, _CHG)])
                return 0

            lax.fori_loop(0, _NCHG // _NBUF, block, 0)

    return pl.kernel(
        body,
        out_type=(jax.ShapeDtypeStruct((_ES, _W), jnp.float32),
                  jax.ShapeDtypeStruct((_ES, _W), jnp.float32)),
        mesh=_mesh(),
        scratch_types=[pltpu.VMEM((_NCHG, _CHG), jnp.int32),
                       pltpu.VMEM((_NBUF, _CHG, _W), jnp.float32),
                       pltpu.SemaphoreType.DMA],
    )(table, sidx3, didx3)


def _sc_scatter(dout, didx3, zeros):
    def body(dout_hbm, didx_hbm, zeros_hbm, acc_hbm, idx_v, rows_v, acc_sh,
             sem, sem2):
        c = lax.axis_index("c")
        s = lax.axis_index("s")
        wid = s * _NC + c
        pltpu.sync_copy(didx_hbm.at[wid], idx_v)
        for k in range(_W // 128):
            pltpu.sync_copy(zeros_hbm, acc_sh.at[pl.ds(s * _RPT, _RPT)])

            @pl.when(s == 0)
            def _():
                pltpu.sync_copy(zeros_hbm.at[pl.ds(0, _REM)],
                                acc_sh.at[pl.ds(_RPT * _NS, _REM)])

            plsc.subcore_barrier()

            def block(t, _):
                j0 = t * _NBUF
                descs = [
                    pltpu.async_copy(
                        dout_hbm.at[pl.ds(wid * _EPW + (j0 + b) * _CH, _CH),
                                    pl.ds(k * 128, 128)], rows_v.at[b], sem)
                    for b in range(_NBUF)
                ]
                adds = []
                for b in range(_NBUF):
                    descs[b].wait()
                    adds.append(pltpu.async_copy(
                        rows_v.at[b], acc_sh.at[idx_v.at[j0 + b]], sem2,
                        add=True))
                for a in adds:
                    a.wait()
                return 0

            lax.fori_loop(0, _NCH // _NBUF, block, 0)
            plsc.subcore_barrier()
            pltpu.sync_copy(acc_sh.at[pl.ds(s * _RPT, _RPT)],
                            acc_hbm.at[c, pl.ds(s * _RPT, _RPT),
                                       pl.ds(k * 128, 128)])

            @pl.when(s == 0)
            def _():
                pltpu.sync_copy(acc_sh.at[pl.ds(_RPT * _NS, _REM)],
                                acc_hbm.at[c, pl.ds(_RPT * _NS, _REM),
                                           pl.ds(k * 128, 128)])

            plsc.subcore_barrier()

    return pl.kernel(
        body,
        out_type=jax.ShapeDtypeStruct((_NC, _N, _W), jnp.float32),
        mesh=_mesh(),
        scratch_types=[pltpu.VMEM((_NCH, _CH), jnp.int32),
                       pltpu.VMEM((_NBUF, _CH, 128), jnp.float32),
                       pltpu.VMEM_SHARED((_N, 128), jnp.float32),
                       pltpu.SemaphoreType.DMA,
                       pltpu.SemaphoreType.DMA],
    )(dout, didx3, zeros)


def _silu(v):
    return v * jax.nn.sigmoid(v)


def _encoder(x, pos, ln_g, ln_b, w0, b0, w1, b1):
    def body(x_ref, pos_ref, g_ref, b_ref, w0_ref, b0_ref, w1_ref, b1_ref,
             out_ref):
        xv = x_ref[...]
        mu = jnp.mean(xv, axis=1, keepdims=True)
        var = jnp.mean((xv - mu) ** 2, axis=1, keepdims=True)
        h = (xv - mu) / jnp.sqrt(var + 1e-5) * g_ref[...] + b_ref[...]
        h = _silu(h @ w0_ref[...] + b0_ref[...])
        h = _silu(h @ w1_ref[...] + b1_ref[...])
        out_ref[...] = jnp.concatenate(
            [h, pos_ref[...], jnp.zeros((_BN, _W - _SH - 3), jnp.float32)],
            axis=1)

    full = lambda shape: pl.BlockSpec(shape, lambda i: (0,) * len(shape))
    return pl.pallas_call(
        body,
        grid=(_N // _BN,),
        in_specs=[pl.BlockSpec((_BN, 128), lambda i: (i, 0)),
                  pl.BlockSpec((_BN, 3), lambda i: (i, 0)),
                  full((1, 128)), full((1, 128)),
                  full((128, _SH // 2)), full((1, _SH // 2)),
                  full((_SH // 2, _SH)), full((1, _SH))],
        out_specs=pl.BlockSpec((_BN, _W), lambda i: (i, 0)),
        out_shape=jax.ShapeDtypeStruct((_N, _W), jnp.float32),
    )(x, pos, ln_g, ln_b, w0, b0, w1, b1)


_WIDTH = np.float32(30.0 / _NRBF)


def _edge_mlp(gs, gd, ea, w1a, w1b, w1c, w1d, b1, w2, b2, w3, b3,
              gw1, gb1, gw2, gb2, hw, hb, xw1, xb1, xw2, xb2):
    def body(gs_ref, gd_ref, ea_ref, w1a_ref, w1b_ref, w1c_ref, w1d_ref,
             b1_ref, w2_ref, b2_ref, w3_ref, b3_ref, gw1_ref, gb1_ref,
             gw2_ref, gb2_ref, hw_ref, hb_ref, xw1_ref, xb1_ref, xw2_ref,
             xb2_ref, out_ref):
        gs_v = gs_ref[...]
        gd_v = gd_ref[...]
        hs = gs_v[:, :_SH]
        hd = gd_v[:, :_SH]
        r = gs_v[:, _SH:_SH + 3] - gd_v[:, _SH:_SH + 3]
        d = jnp.sqrt(jnp.sum(r * r, axis=1, keepdims=True) + 1e-12)
        centers = lax.broadcasted_iota(jnp.int32, (1, _NRBF), 1).astype(
            jnp.float32) * np.float32(30.0 / (_NRBF - 1))
        z = (d - centers) / (_WIDTH + np.float32(1e-8))
        rbf = jnp.exp(-(z * z))
        m = (hs @ w1a_ref[...] + hd @ w1b_ref[...] + rbf @ w1c_ref[...]
             + ea_ref[...] @ w1d_ref[...] + b1_ref[...])
        m = _silu(m)
        m = _silu(m @ w2_ref[...] + b2_ref[...])
        m = _silu(m @ w3_ref[...] + b3_ref[...])
        g1 = jax.nn.relu(m @ gw1_ref[...] + gb1_ref[...])
        a = jax.nn.sigmoid(g1 @ gw2_ref[...] + gb2_ref[...])
        m = m * a
        dh = _silu(m @ hw_ref[...] + hb_ref[...])
        coeff = (_silu(m @ xw1_ref[...] + xb1_ref[...]) @ xw2_ref[...]
                 + xb2_ref[...]) * np.float32(0.08)
        out_ref[...] = jnp.concatenate(
            [dh, r * coeff, jnp.zeros((_BE, _W - _SH - 3), jnp.float32)],
            axis=1)

    full = lambda shape: pl.BlockSpec(shape, lambda i: (0,) * len(shape))
    return pl.pallas_call(
        body,
        grid=(_ES // _BE,),
        in_specs=[pl.BlockSpec((_BE, _W), lambda i: (i, 0)),
                  pl.BlockSpec((_BE, _W), lambda i: (i, 0)),
                  pl.BlockSpec((_BE, 16), lambda i: (i, 0)),
                  full((_SH, 320)), full((_SH, 320)), full((_NRBF, 320)),
                  full((16, 320)), full((1, 320)),
                  full((320, 160)), full((1, 160)),
                  full((160, 128)), full((1, 128)),
                  full((128, 64)), full((1, 64)),
                  full((64, 1)), full((1, 1)),
                  full((128, _SH)), full((1, _SH)),
                  full((128, 32)), full((1, 32)),
                  full((32, 1)), full((1, 1))],
        out_specs=pl.BlockSpec((_BE, _W), lambda i: (i, 0)),
        out_shape=jax.ShapeDtypeStruct((_ES, _W), jnp.float32),
    )(gs, gd, ea, w1a, w1b, w1c, w1d, b1, w2, b2, w3, b3,
      gw1, gb1, gw2, gb2, hw, hb, xw1, xb1, xw2, xb2)


def _node_update(table, accs, lng, lnb, alpha_i):
    na = len(accs)

    def body(*refs):
        t_ref = refs[0]
        acc_refs = refs[1:1 + na]
        g_ref, b_ref, al_ref, out_ref = refs[1 + na:]
        t = t_ref[...]
        asum = acc_refs[0][...]
        for ar in acc_refs[1:]:
            asum = asum + ar[...]
        sg = jax.nn.sigmoid(al_ref[0, 0])
        u = t[:, :_SH] + sg * asum[:, :_SH]
        mu = jnp.mean(u, axis=1, keepdims=True)
        var = jnp.mean((u - mu) ** 2, axis=1, keepdims=True)
        hn = (u - mu) / jnp.sqrt(var + 1e-5) * g_ref[...] + b_ref[...]
        xcn = t[:, _SH:_SH + 3] + asum[:, _SH:_SH + 3]
        out_ref[...] = jnp.concatenate(
            [hn, xcn, jnp.zeros((_BN, _W - _SH - 3), jnp.float32)], axis=1)

    full = lambda shape: pl.BlockSpec(shape, lambda i: (0,) * len(shape))
    return pl.pallas_call(
        body,
        grid=(_N // _BN,),
        in_specs=[pl.BlockSpec((_BN, _W), lambda i: (i, 0))] * (1 + na)
                 + [full((1, _SH)), full((1, _SH)), full((1, 1))],
        out_specs=pl.BlockSpec((_BN, _W), lambda i: (i, 0)),
        out_shape=jax.ShapeDtypeStruct((_N, _W), jnp.float32),
    )(table, *accs, lng, lnb, alpha_i)


def kernel(x, pos, edge_attr, edge_index, ln_g, ln_b, W0, b0, W1, b1,
           Ew1, Eb1, Ew2, Eb2, Ew3, Eb3, Gw1, Gb1, Gw2, Gb2,
           Hw, Hb, Xw1, Xb1, Xw2, Xb2, LNg, LNb, alpha):
    sidxg = edge_index[0].reshape(_NSPLIT, _NW, _NCHG, _CHG)
    didxg = edge_index[1].reshape(_NSPLIT, _NW, _NCHG, _CHG)
    didx = edge_index[1].reshape(_NSPLIT, _NW, _NCH, _CH)
    ea = edge_attr.reshape(_NSPLIT, _ES, edge_attr.shape[1])
    zeros = jnp.zeros((_RPT, 128), jnp.float32)
    table = _encoder(x, pos, ln_g.reshape(1, -1), ln_b.reshape(1, -1),
                     W0, b0.reshape(1, -1), W1, b1.reshape(1, -1))
    for i in range(Ew1.shape[0]):
        w1a = Ew1[i, :_SH]
        w1b = Ew1[i, _SH:2 * _SH]
        w1c = Ew1[i, 2 * _SH:2 * _SH + _NRBF]
        w1d = Ew1[i, 2 * _SH + _NRBF:]
        accs = []
        for sp in range(_NSPLIT):
            gs, gd = _sc_gather(table, sidxg[sp], didxg[sp])
            dout = _edge_mlp(gs, gd, ea[sp], w1a, w1b, w1c, w1d,
                             Eb1[i].reshape(1, -1), Ew2[i],
                             Eb2[i].reshape(1, -1),
                             Ew3[i], Eb3[i].reshape(1, -1),
                             Gw1[i], Gb1[i].reshape(1, -1),
                             Gw2[i], Gb2[i].reshape(1, -1),
                             Hw[i], Hb[i].reshape(1, -1),
                             Xw1[i], Xb1[i].reshape(1, -1),
                             Xw2[i], Xb2[i].reshape(1, -1))
            acc = _sc_scatter(dout, didx[sp], zeros)
            accs.extend([acc[0], acc[1]])
        table = _node_update(table, accs, LNg[i].reshape(1, -1),
                             LNb[i].reshape(1, -1), alpha[i].reshape(1, 1))
    return table[:, :_SH], table[:, _SH:_SH + 3]
